# Initial kernel scaffold; baseline (speedup 1.0000x reference)
import jax
import jax.numpy as jnp
from jax.experimental import pallas as pl


def kernel(hidden_states, W):
    raise NotImplementedError("write your pallas kernel here")


# speedup vs baseline: 1.1723x; 1.1723x over previous
import functools

import jax
import jax.numpy as jnp
from jax.experimental import pallas as pl

HIDDEN = 4096
NUM_EXPERTS = 64
TOP_K = 8
BLOCK_T = 512


def _router_block(x_ref, w_ref, tw_ref, ti_ref):
    x = x_ref[...]
    w = w_ref[...]
    logits = jax.lax.dot_general(
        x, w, (((1,), (1,)), ((), ())), preferred_element_type=jnp.float32
    )
    scores = jnp.maximum(logits, 0.0)

    bt = scores.shape[0]
    lane = jax.lax.broadcasted_iota(jnp.int32, (bt, NUM_EXPERTS), 1)
    vals = []
    idxs = []
    s = scores
    for _ in range(TOP_K):
        m = jnp.max(s, axis=1, keepdims=True)
        i = jnp.min(jnp.where(s == m, lane, NUM_EXPERTS), axis=1, keepdims=True)
        vals.append(m)
        idxs.append(i)
        s = jnp.where(lane == i, -1.0, s)
    tw = jnp.concatenate(vals, axis=1)
    ti = jnp.concatenate(idxs, axis=1)
    tw = tw / (jnp.sum(tw, axis=1, keepdims=True) + 1e-6)
    tw_ref[...] = tw
    ti_ref[...] = ti


@jax.jit
def kernel(hidden_states, W):
    tokens = hidden_states.shape[0]
    grid = (tokens // BLOCK_T,)
    tw, ti = pl.pallas_call(
        _router_block,
        grid=grid,
        in_specs=[
            pl.BlockSpec((BLOCK_T, HIDDEN), lambda i: (i, 0)),
            pl.BlockSpec((NUM_EXPERTS, HIDDEN), lambda i: (0, 0)),
        ],
        out_specs=[
            pl.BlockSpec((BLOCK_T, TOP_K), lambda i: (i, 0)),
            pl.BlockSpec((BLOCK_T, TOP_K), lambda i: (i, 0)),
        ],
        out_shape=[
            jax.ShapeDtypeStruct((tokens, TOP_K), jnp.float32),
            jax.ShapeDtypeStruct((tokens, TOP_K), jnp.int32),
        ],
    )(hidden_states, W)
    return tw, ti


# speedup vs baseline: 1.3159x; 1.1224x over previous
import functools

import jax
import jax.numpy as jnp
from jax.experimental import pallas as pl

HIDDEN = 4096
NUM_EXPERTS = 64
TOP_K = 8
BLOCK_T = 512


def _router_block(x_ref, w_ref, tw_ref, ti_ref):
    x = x_ref[...]
    w = w_ref[...]
    logits = jax.lax.dot_general(
        x, w, (((1,), (1,)), ((), ())), preferred_element_type=jnp.float32
    )
    scores = jnp.maximum(logits, 0.0)

    bt = scores.shape[0]
    lanef = jax.lax.broadcasted_iota(jnp.int32, (bt, NUM_EXPERTS), 1).astype(
        jnp.float32
    )
    vals = []
    idxs = []
    s = scores
    for _ in range(TOP_K):
        m = jnp.max(s, axis=1, keepdims=True)
        i = jnp.min(jnp.where(s == m, lanef, float(NUM_EXPERTS)), axis=1, keepdims=True)
        vals.append(m)
        idxs.append(i)
        s = jnp.where(lanef == i, -1.0, s)
    tw = jnp.concatenate(vals, axis=1)
    ti = jnp.concatenate(idxs, axis=1).astype(jnp.int32)
    tw = tw / (jnp.sum(tw, axis=1, keepdims=True) + 1e-6)
    tw_ref[...] = tw
    ti_ref[...] = ti


@jax.jit
def kernel(hidden_states, W):
    tokens = hidden_states.shape[0]
    grid = (tokens // BLOCK_T,)
    tw, ti = pl.pallas_call(
        _router_block,
        grid=grid,
        in_specs=[
            pl.BlockSpec((BLOCK_T, HIDDEN), lambda i: (i, 0)),
            pl.BlockSpec((NUM_EXPERTS, HIDDEN), lambda i: (0, 0)),
        ],
        out_specs=[
            pl.BlockSpec((BLOCK_T, TOP_K), lambda i: (i, 0)),
            pl.BlockSpec((BLOCK_T, TOP_K), lambda i: (i, 0)),
        ],
        out_shape=[
            jax.ShapeDtypeStruct((tokens, TOP_K), jnp.float32),
            jax.ShapeDtypeStruct((tokens, TOP_K), jnp.int32),
        ],
    )(hidden_states, W)
    return tw, ti


# speedup vs baseline: 1.7725x; 1.3471x over previous
import functools

import jax
import jax.numpy as jnp
from jax.experimental import pallas as pl

HIDDEN = 4096
NUM_EXPERTS = 64
TOP_K = 8
BLOCK_T = 512


def _router_block(x_ref, w_ref, tw_ref, ti_ref):
    x = x_ref[...]
    w = w_ref[...]
    logits = jax.lax.dot_general(
        x, w, (((1,), (1,)), ((), ())), preferred_element_type=jnp.float32
    )
    scores = jnp.maximum(logits, 0.0)

    bt = scores.shape[0]
    s = scores.T
    lanef = jax.lax.broadcasted_iota(jnp.int32, (NUM_EXPERTS, bt), 0).astype(
        jnp.float32
    )
    vals = []
    idxs = []
    for _ in range(TOP_K):
        m = jnp.max(s, axis=0, keepdims=True)
        i = jnp.min(
            jnp.where(s == m, lanef, float(NUM_EXPERTS)), axis=0, keepdims=True
        )
        vals.append(m)
        idxs.append(i)
        s = jnp.where(lanef == i, -1.0, s)
    tw = jnp.concatenate(vals, axis=0)
    ti = jnp.concatenate(idxs, axis=0).astype(jnp.int32)
    tw = tw / (jnp.sum(tw, axis=0, keepdims=True) + 1e-6)
    tw_ref[...] = tw
    ti_ref[...] = ti


@jax.jit
def kernel(hidden_states, W):
    tokens = hidden_states.shape[0]
    grid = (tokens // BLOCK_T,)
    tw_t, ti_t = pl.pallas_call(
        _router_block,
        grid=grid,
        in_specs=[
            pl.BlockSpec((BLOCK_T, HIDDEN), lambda i: (i, 0)),
            pl.BlockSpec((NUM_EXPERTS, HIDDEN), lambda i: (0, 0)),
        ],
        out_specs=[
            pl.BlockSpec((TOP_K, BLOCK_T), lambda i: (0, i)),
            pl.BlockSpec((TOP_K, BLOCK_T), lambda i: (0, i)),
        ],
        out_shape=[
            jax.ShapeDtypeStruct((TOP_K, tokens), jnp.float32),
            jax.ShapeDtypeStruct((TOP_K, tokens), jnp.int32),
        ],
    )(hidden_states, W)
    return tw_t.T, ti_t.T


# speedup vs baseline: 1.8237x; 1.0288x over previous
import functools

import jax
import jax.numpy as jnp
from jax.experimental import pallas as pl

HIDDEN = 4096
NUM_EXPERTS = 64
TOP_K = 8
BLOCK_T = 1024


def _router_block(x_ref, w_ref, tw_ref, ti_ref):
    x = x_ref[...]
    w = w_ref[...]
    logits = jax.lax.dot_general(
        x, w, (((1,), (1,)), ((), ())), preferred_element_type=jnp.float32
    )
    scores = jnp.maximum(logits, 0.0)

    bt = scores.shape[0]
    s = scores.T
    lanef = jax.lax.broadcasted_iota(jnp.int32, (NUM_EXPERTS, bt), 0).astype(
        jnp.float32
    )
    vals = []
    idxs = []
    for _ in range(TOP_K):
        m = jnp.max(s, axis=0, keepdims=True)
        i = jnp.min(
            jnp.where(s == m, lanef, float(NUM_EXPERTS)), axis=0, keepdims=True
        )
        vals.append(m)
        idxs.append(i)
        s = jnp.where(lanef == i, -1.0, s)
    tw = jnp.concatenate(vals, axis=0)
    ti = jnp.concatenate(idxs, axis=0).astype(jnp.int32)
    tw = tw / (jnp.sum(tw, axis=0, keepdims=True) + 1e-6)
    tw_ref[...] = tw
    ti_ref[...] = ti


@jax.jit
def kernel(hidden_states, W):
    tokens = hidden_states.shape[0]
    grid = (tokens // BLOCK_T,)
    tw_t, ti_t = pl.pallas_call(
        _router_block,
        grid=grid,
        in_specs=[
            pl.BlockSpec((BLOCK_T, HIDDEN), lambda i: (i, 0)),
            pl.BlockSpec((NUM_EXPERTS, HIDDEN), lambda i: (0, 0)),
        ],
        out_specs=[
            pl.BlockSpec((TOP_K, BLOCK_T), lambda i: (0, i)),
            pl.BlockSpec((TOP_K, BLOCK_T), lambda i: (0, i)),
        ],
        out_shape=[
            jax.ShapeDtypeStruct((TOP_K, tokens), jnp.float32),
            jax.ShapeDtypeStruct((TOP_K, tokens), jnp.int32),
        ],
    )(hidden_states, W)
    return tw_t.T, ti_t.T
